# baseline (speedup 1.0000x reference)
import functools

import jax
import jax.numpy as jnp
import numpy as np
from jax import lax
from jax.experimental import pallas as pl
from jax.experimental.pallas import tpu as pltpu
from jax.experimental.pallas import tpu_sc as plsc

BATCH = 1024
INPUT_SIZE = 1024
NUM_NEURONS = 4096

NC = 2
NS = 16
L = 16
NW = NC * NS
ROWS_PER_TILE = BATCH // NW
GROUP = 8
NGROUPS = ROWS_PER_TILE // GROUP
NCHUNK = NUM_NEURONS // L

_C_TABLE = np.array([
    [0, 0, 0, 0],
    [0, 0, 0, 1],
    [0, 1, 0, -1],
    [0, 1, 0, 0],
    [0, 0, 1, -1],
    [0, 0, 1, 0],
    [0, 1, 1, -2],
    [0, 1, 1, -1],
    [1, -1, -1, 1],
    [1, -1, -1, 2],
    [1, 0, -1, 0],
    [1, 0, -1, 1],
    [1, -1, 0, 0],
    [1, -1, 0, 1],
    [1, 0, 0, -1],
    [1, 0, 0, 0],
], dtype=np.float32)


def _coef_body(gwt_ref, idxt_ref, ct_ref, coef_ref, ia_ref, ib_ref):
    gw = gwt_ref[...]
    m = jnp.max(gw, axis=0, keepdims=True)
    e = jnp.exp(gw - m)
    s = jnp.sum(e, axis=0, keepdims=True)
    sw = e / s
    k4 = lax.dot_general(
        ct_ref[...], sw,
        dimension_numbers=(((1,), (0,)), ((), ())),
        preferred_element_type=jnp.float32)
    coef_ref[...] = jnp.concatenate(
        [k4, jnp.zeros((4, NUM_NEURONS), jnp.float32)], axis=0)
    ia_ref[...] = idxt_ref[0, :]
    ib_ref[...] = idxt_ref[1, :]


def _sc_body(x_hbm, ia_hbm, ib_hbm, coef_hbm, out_hbm,
             ia_v, ib_v, coef_v, rows0_v, rows1_v, out0_v, out1_v,
             sem_ia, sem_ib, sem_cf, sem_r0, sem_r1, sem_o0, sem_o1):
    wid = lax.axis_index("s") * NC + lax.axis_index("c")
    row0 = wid * ROWS_PER_TILE
    rows_bufs = [rows0_v, rows1_v]
    out_bufs = [out0_v, out1_v]
    rows_sems = [sem_r0, sem_r1]
    out_sems = [sem_o0, sem_o1]

    d_ia = pltpu.async_copy(ia_hbm, ia_v, sem_ia)
    d_ib = pltpu.async_copy(ib_hbm, ib_v, sem_ib)
    d_cf = pltpu.async_copy(coef_hbm, coef_v, sem_cf)
    d_rows = [None, None]
    d_rows[0] = pltpu.async_copy(
        x_hbm.at[pl.ds(row0, GROUP)], rows0_v, sem_r0)
    d_out = [None, None]
    d_ia.wait()
    d_ib.wait()
    d_cf.wait()

    for g in range(NGROUPS):
        cur = g % 2
        nxt = 1 - cur
        base = row0 + g * GROUP
        d_rows[cur].wait()
        if g + 1 < NGROUPS:
            d_rows[nxt] = pltpu.async_copy(
                x_hbm.at[pl.ds(base + GROUP, GROUP)],
                rows_bufs[nxt], rows_sems[nxt])
        if d_out[cur] is not None:
            d_out[cur].wait()
        rows_v = rows_bufs[cur]
        out_v = out_bufs[cur]

        @plsc.parallel_loop(0, NCHUNK, unroll=4)
        def _chunk(c):
            off = c * L
            ia16 = ia_v[pl.ds(off, L)]
            ib16 = ib_v[pl.ds(off, L)]
            k0 = coef_v[0, pl.ds(off, L)]
            ka = coef_v[1, pl.ds(off, L)]
            kb = coef_v[2, pl.ds(off, L)]
            kab = coef_v[3, pl.ds(off, L)]
            avs = []
            bvs = []
            for r in range(GROUP):
                r16 = jnp.full((L,), r, jnp.int32)
                avs.append(plsc.load_gather(rows_v, [r16, ia16]))
                bvs.append(plsc.load_gather(rows_v, [r16, ib16]))
            for r in range(GROUP):
                a = avs[r]
                b = bvs[r]
                out_v[r, pl.ds(off, L)] = (k0 + a * ka) + b * (kb + a * kab)

        d_out[cur] = pltpu.async_copy(
            out_v, out_hbm.at[pl.ds(base, GROUP)], out_sems[cur])

    d_out[0].wait()
    d_out[1].wait()


_sc_kernel = functools.partial(
    pl.kernel,
    out_type=jax.ShapeDtypeStruct((BATCH, NUM_NEURONS), jnp.float32),
    mesh=plsc.VectorSubcoreMesh(core_axis_name="c", subcore_axis_name="s",
                                num_cores=NC, num_subcores=NS),
    scratch_types=[
        pltpu.VMEM((NUM_NEURONS,), jnp.int32),
        pltpu.VMEM((NUM_NEURONS,), jnp.int32),
        pltpu.VMEM((8, NUM_NEURONS), jnp.float32),
        pltpu.VMEM((GROUP, INPUT_SIZE), jnp.float32),
        pltpu.VMEM((GROUP, INPUT_SIZE), jnp.float32),
        pltpu.VMEM((GROUP, NUM_NEURONS), jnp.float32),
        pltpu.VMEM((GROUP, NUM_NEURONS), jnp.float32),
        pltpu.SemaphoreType.DMA,
        pltpu.SemaphoreType.DMA,
        pltpu.SemaphoreType.DMA,
        pltpu.SemaphoreType.DMA,
        pltpu.SemaphoreType.DMA,
        pltpu.SemaphoreType.DMA,
        pltpu.SemaphoreType.DMA,
    ],
    compiler_params=pltpu.CompilerParams(use_tc_tiling_on_sc=True,
                                         needs_layout_passes=False),
)(_sc_body)


def kernel(x, input_idx, gate_weights):
    ct = jnp.asarray(_C_TABLE.T)
    coef, ia, ib = pl.pallas_call(
        _coef_body,
        out_shape=[
            jax.ShapeDtypeStruct((8, NUM_NEURONS), jnp.float32),
            jax.ShapeDtypeStruct((NUM_NEURONS,), jnp.int32),
            jax.ShapeDtypeStruct((NUM_NEURONS,), jnp.int32),
        ],
    )(gate_weights.T, input_idx.T, ct)
    return _sc_kernel(x, ia, ib, coef)


# speedup vs baseline: 1.2390x; 1.2390x over previous
import functools

import jax
import jax.numpy as jnp
import numpy as np
from jax import lax
from jax.experimental import pallas as pl
from jax.experimental.pallas import tpu as pltpu
from jax.experimental.pallas import tpu_sc as plsc

BATCH = 1024
INPUT_SIZE = 1024
NUM_NEURONS = 4096

NC = 2
NS = 16
L = 16
NW = NC * NS
ROWS_PER_TILE = BATCH // NW
GROUP = 8
NGROUPS = ROWS_PER_TILE // GROUP
NCHUNK = NUM_NEURONS // L

_C_TABLE = np.array([
    [0, 0, 0, 0],
    [0, 0, 0, 1],
    [0, 1, 0, -1],
    [0, 1, 0, 0],
    [0, 0, 1, -1],
    [0, 0, 1, 0],
    [0, 1, 1, -2],
    [0, 1, 1, -1],
    [1, -1, -1, 1],
    [1, -1, -1, 2],
    [1, 0, -1, 0],
    [1, 0, -1, 1],
    [1, -1, 0, 0],
    [1, -1, 0, 1],
    [1, 0, 0, -1],
    [1, 0, 0, 0],
], dtype=np.float32)


def _coef_body(gwt_ref, idxt_ref, ct_ref, coef_ref, ia_ref, ib_ref):
    gw = gwt_ref[...]
    m = jnp.max(gw, axis=0, keepdims=True)
    e = jnp.exp(gw - m)
    s = jnp.sum(e, axis=0, keepdims=True)
    sw = e / s
    k4 = lax.dot_general(
        ct_ref[...], sw,
        dimension_numbers=(((1,), (0,)), ((), ())),
        preferred_element_type=jnp.float32)
    coef_ref[...] = jnp.concatenate(
        [k4, jnp.zeros((4, NUM_NEURONS), jnp.float32)], axis=0)
    ia_ref[...] = idxt_ref[0, :]
    ib_ref[...] = idxt_ref[1, :]


def _sc_body(x_hbm, ia_hbm, ib_hbm, coef_hbm, out_hbm,
             ia_v, ib_v, coef_v, rows0_v, rows1_v, out0_v, out1_v,
             sem_ia, sem_ib, sem_cf, sem_r0, sem_r1, sem_o0, sem_o1):
    wid = lax.axis_index("s") * NC + lax.axis_index("c")
    row0 = wid * ROWS_PER_TILE
    rows_bufs = [rows0_v, rows1_v]
    out_bufs = [out0_v, out1_v]
    rows_sems = [sem_r0, sem_r1]
    out_sems = [sem_o0, sem_o1]

    d_ia = pltpu.async_copy(ia_hbm, ia_v, sem_ia)
    d_ib = pltpu.async_copy(ib_hbm, ib_v, sem_ib)
    d_cf = pltpu.async_copy(coef_hbm, coef_v, sem_cf)
    d_rows = [None, None]
    d_rows[0] = pltpu.async_copy(
        x_hbm.at[pl.ds(row0, GROUP)], rows0_v, sem_r0)
    d_out = [None, None]
    d_ia.wait()
    d_ib.wait()
    d_cf.wait()

    for g in range(NGROUPS):
        cur = g % 2
        nxt = 1 - cur
        base = row0 + g * GROUP
        d_rows[cur].wait()
        if g + 1 < NGROUPS:
            d_rows[nxt] = pltpu.async_copy(
                x_hbm.at[pl.ds(base + GROUP, GROUP)],
                rows_bufs[nxt], rows_sems[nxt])
        if d_out[cur] is not None:
            d_out[cur].wait()
        rows_v = rows_bufs[cur]
        out_v = out_bufs[cur]

        @plsc.parallel_loop(0, NCHUNK, unroll=2)
        def _chunk(c):
            off = c * L
            ia16 = ia_v[pl.ds(off, L)]
            ib16 = ib_v[pl.ds(off, L)]
            k0 = coef_v[0, pl.ds(off, L)]
            ka = coef_v[1, pl.ds(off, L)]
            kb = coef_v[2, pl.ds(off, L)]
            kab = coef_v[3, pl.ds(off, L)]
            avs = []
            bvs = []
            for r in range(GROUP):
                r16 = jnp.full((L,), r, jnp.int32)
                avs.append(plsc.load_gather(rows_v, [r16, ia16]))
                bvs.append(plsc.load_gather(rows_v, [r16, ib16]))
            for r in range(GROUP):
                a = avs[r]
                b = bvs[r]
                out_v[r, pl.ds(off, L)] = (k0 + a * ka) + b * (kb + a * kab)

        d_out[cur] = pltpu.async_copy(
            out_v, out_hbm.at[pl.ds(base, GROUP)], out_sems[cur])

    d_out[0].wait()
    d_out[1].wait()


_sc_kernel = functools.partial(
    pl.kernel,
    out_type=jax.ShapeDtypeStruct((BATCH, NUM_NEURONS), jnp.float32),
    mesh=plsc.VectorSubcoreMesh(core_axis_name="c", subcore_axis_name="s",
                                num_cores=NC, num_subcores=NS),
    scratch_types=[
        pltpu.VMEM((NUM_NEURONS,), jnp.int32),
        pltpu.VMEM((NUM_NEURONS,), jnp.int32),
        pltpu.VMEM((8, NUM_NEURONS), jnp.float32),
        pltpu.VMEM((GROUP, INPUT_SIZE), jnp.float32),
        pltpu.VMEM((GROUP, INPUT_SIZE), jnp.float32),
        pltpu.VMEM((GROUP, NUM_NEURONS), jnp.float32),
        pltpu.VMEM((GROUP, NUM_NEURONS), jnp.float32),
        pltpu.SemaphoreType.DMA,
        pltpu.SemaphoreType.DMA,
        pltpu.SemaphoreType.DMA,
        pltpu.SemaphoreType.DMA,
        pltpu.SemaphoreType.DMA,
        pltpu.SemaphoreType.DMA,
        pltpu.SemaphoreType.DMA,
    ],
    compiler_params=pltpu.CompilerParams(use_tc_tiling_on_sc=True,
                                         needs_layout_passes=False),
)(_sc_body)


def kernel(x, input_idx, gate_weights):
    ct = jnp.asarray(_C_TABLE.T)
    coef, ia, ib = pl.pallas_call(
        _coef_body,
        out_shape=[
            jax.ShapeDtypeStruct((8, NUM_NEURONS), jnp.float32),
            jax.ShapeDtypeStruct((NUM_NEURONS,), jnp.int32),
            jax.ShapeDtypeStruct((NUM_NEURONS,), jnp.int32),
        ],
    )(gate_weights.T, input_idx.T, ct)
    return _sc_kernel(x, ia, ib, coef)


# speedup vs baseline: 1.2397x; 1.0006x over previous
import functools

import jax
import jax.numpy as jnp
import numpy as np
from jax import lax
from jax.experimental import pallas as pl
from jax.experimental.pallas import tpu as pltpu
from jax.experimental.pallas import tpu_sc as plsc

BATCH = 1024
INPUT_SIZE = 1024
NUM_NEURONS = 4096

NC = 2
NS = 16
L = 16
NW = NC * NS
ROWS_PER_TILE = BATCH // NW
GROUP = 8
NGROUPS = ROWS_PER_TILE // GROUP
NCHUNK = NUM_NEURONS // L

_C_TABLE = np.array([
    [0, 0, 0, 0],
    [0, 0, 0, 1],
    [0, 1, 0, -1],
    [0, 1, 0, 0],
    [0, 0, 1, -1],
    [0, 0, 1, 0],
    [0, 1, 1, -2],
    [0, 1, 1, -1],
    [1, -1, -1, 1],
    [1, -1, -1, 2],
    [1, 0, -1, 0],
    [1, 0, -1, 1],
    [1, -1, 0, 0],
    [1, -1, 0, 1],
    [1, 0, 0, -1],
    [1, 0, 0, 0],
], dtype=np.float32)


def _coef_body(gwt_ref, idxt_ref, ct_ref, coef_ref, ia_ref, ib_ref,
               coef_v, ia_v, ib_v, sem_cf, sem_ia, sem_ib):
    gw = gwt_ref[...]
    m = jnp.max(gw, axis=0, keepdims=True)
    e = jnp.exp(gw - m)
    s = jnp.sum(e, axis=0, keepdims=True)
    sw = e / s
    k4 = lax.dot_general(
        ct_ref[...], sw,
        dimension_numbers=(((1,), (0,)), ((), ())),
        preferred_element_type=jnp.float32)
    coef_v[...] = jnp.concatenate(
        [k4, jnp.zeros((4, NUM_NEURONS), jnp.float32)], axis=0)
    ia_v[...] = idxt_ref[0, :]
    ib_v[...] = idxt_ref[1, :]
    d_cf = pltpu.async_copy(coef_v, coef_ref, sem_cf)
    d_ia = pltpu.async_copy(ia_v, ia_ref, sem_ia)
    d_ib = pltpu.async_copy(ib_v, ib_ref, sem_ib)
    d_cf.wait()
    d_ia.wait()
    d_ib.wait()


def _sc_body(x_hbm, ia_hbm, ib_hbm, coef_hbm, out_hbm,
             ia_v, ib_v, coef_v, rows0_v, rows1_v, out0_v, out1_v,
             sem_ia, sem_ib, sem_cf, sem_r0, sem_r1, sem_o0, sem_o1):
    wid = lax.axis_index("s") * NC + lax.axis_index("c")
    row0 = wid * ROWS_PER_TILE
    rows_bufs = [rows0_v, rows1_v]
    out_bufs = [out0_v, out1_v]
    rows_sems = [sem_r0, sem_r1]
    out_sems = [sem_o0, sem_o1]

    d_ia = pltpu.async_copy(ia_hbm, ia_v, sem_ia)
    d_ib = pltpu.async_copy(ib_hbm, ib_v, sem_ib)
    d_cf = pltpu.async_copy(coef_hbm, coef_v, sem_cf)
    d_rows = [None, None]
    d_rows[0] = pltpu.async_copy(
        x_hbm.at[pl.ds(row0, GROUP)], rows0_v, sem_r0)
    d_out = [None, None]
    d_ia.wait()
    d_ib.wait()
    d_cf.wait()

    for g in range(NGROUPS):
        cur = g % 2
        nxt = 1 - cur
        base = row0 + g * GROUP
        d_rows[cur].wait()
        if g + 1 < NGROUPS:
            d_rows[nxt] = pltpu.async_copy(
                x_hbm.at[pl.ds(base + GROUP, GROUP)],
                rows_bufs[nxt], rows_sems[nxt])
        if d_out[cur] is not None:
            d_out[cur].wait()
        rows_v = rows_bufs[cur]
        out_v = out_bufs[cur]

        @plsc.parallel_loop(0, NCHUNK, unroll=2)
        def _chunk(c):
            off = c * L
            ia16 = ia_v[pl.ds(off, L)]
            ib16 = ib_v[pl.ds(off, L)]
            k0 = coef_v[0, pl.ds(off, L)]
            ka = coef_v[1, pl.ds(off, L)]
            kb = coef_v[2, pl.ds(off, L)]
            kab = coef_v[3, pl.ds(off, L)]
            avs = []
            bvs = []
            for r in range(GROUP):
                r16 = jnp.full((L,), r, jnp.int32)
                avs.append(plsc.load_gather(rows_v, [r16, ia16]))
                bvs.append(plsc.load_gather(rows_v, [r16, ib16]))
            for r in range(GROUP):
                a = avs[r]
                b = bvs[r]
                out_v[r, pl.ds(off, L)] = (k0 + a * ka) + b * (kb + a * kab)

        d_out[cur] = pltpu.async_copy(
            out_v, out_hbm.at[pl.ds(base, GROUP)], out_sems[cur])

    d_out[0].wait()
    d_out[1].wait()


_sc_kernel = functools.partial(
    pl.kernel,
    out_type=jax.ShapeDtypeStruct((BATCH, NUM_NEURONS), jnp.float32),
    mesh=plsc.VectorSubcoreMesh(core_axis_name="c", subcore_axis_name="s",
                                num_cores=NC, num_subcores=NS),
    scratch_types=[
        pltpu.VMEM((NUM_NEURONS,), jnp.int32),
        pltpu.VMEM((NUM_NEURONS,), jnp.int32),
        pltpu.VMEM((8, NUM_NEURONS), jnp.float32),
        pltpu.VMEM((GROUP, INPUT_SIZE), jnp.float32),
        pltpu.VMEM((GROUP, INPUT_SIZE), jnp.float32),
        pltpu.VMEM((GROUP, NUM_NEURONS), jnp.float32),
        pltpu.VMEM((GROUP, NUM_NEURONS), jnp.float32),
        pltpu.SemaphoreType.DMA,
        pltpu.SemaphoreType.DMA,
        pltpu.SemaphoreType.DMA,
        pltpu.SemaphoreType.DMA,
        pltpu.SemaphoreType.DMA,
        pltpu.SemaphoreType.DMA,
        pltpu.SemaphoreType.DMA,
    ],
    compiler_params=pltpu.CompilerParams(use_tc_tiling_on_sc=True,
                                         needs_layout_passes=False),
)(_sc_body)


def kernel(x, input_idx, gate_weights):
    ct = jnp.asarray(_C_TABLE.T)
    coef, ia, ib = pl.pallas_call(
        _coef_body,
        out_shape=[
            jax.ShapeDtypeStruct((8, NUM_NEURONS), jnp.float32),
            jax.ShapeDtypeStruct((NUM_NEURONS,), jnp.int32),
            jax.ShapeDtypeStruct((NUM_NEURONS,), jnp.int32),
        ],
        out_specs=[
            pl.BlockSpec(memory_space=pl.ANY),
            pl.BlockSpec(memory_space=pl.ANY),
            pl.BlockSpec(memory_space=pl.ANY),
        ],
        scratch_shapes=[
            pltpu.VMEM((8, NUM_NEURONS), jnp.float32),
            pltpu.VMEM((NUM_NEURONS,), jnp.int32),
            pltpu.VMEM((NUM_NEURONS,), jnp.int32),
            pltpu.SemaphoreType.DMA,
            pltpu.SemaphoreType.DMA,
            pltpu.SemaphoreType.DMA,
        ],
    )(gate_weights.T, input_idx.T, ct)
    return _sc_kernel(x, ia, ib, coef)
